# Initial kernel scaffold; baseline (speedup 1.0000x reference)
import jax
import jax.numpy as jnp
from jax.experimental import pallas as pl


def kernel(x, mask, h0, c0, W_ih, W_hh, b_ih, b_hh):
    raise NotImplementedError("write your pallas kernel here")


# speedup vs baseline: 2.6524x; 2.6524x over previous
import functools

import jax
import jax.numpy as jnp
from jax import lax
from jax.experimental import pallas as pl
from jax.experimental.pallas import tpu as pltpu
from jax.experimental.pallas import tpu_sc as plsc

B = 16384
D = 128
H = 128
G = 4 * H
BLK = 1024
NB = B // BLK
MROWS = 128
MCOLS = B // MROWS


def _tc_body(x_r, h_r, c_r, mrow_r, m2d_r, wih_r, whh_r, b1_r, b2_r,
             xo_r, hm_r, cm_r, pos_r, valid_s):
    i = pl.program_id(0)

    @pl.when(i == 0)
    def _compute_pos():
        mf = m2d_r[...].astype(jnp.float32)
        row = lax.broadcasted_iota(jnp.int32, (MROWS, MCOLS), 0).astype(jnp.float32)
        col = lax.broadcasted_iota(jnp.int32, (MROWS, MCOLS), 1).astype(jnp.float32)
        lt = (row <= col).astype(jnp.float32)
        sl = (col < row).astype(jnp.float32)
        t = lax.dot_general(mf, lt, (((1,), (0,)), ((), ())),
                            preferred_element_type=jnp.float32)
        rowsum = t[:, MCOLS - 1:MCOLS]
        rowoff = lax.dot_general(sl, rowsum, (((1,), (0,)), ((), ())),
                                 preferred_element_type=jnp.float32)
        a = t + rowoff
        valid = jnp.sum(mf)
        valid_s[0] = valid
        bidx = row * float(MCOLS) + col
        posf = jnp.where(mf > 0.0, a - 1.0, valid + bidx - a)
        pos_r[...] = posf.astype(jnp.int32)

    valid = valid_s[0]
    special = jnp.logical_or(valid == 0.0, valid == float(B))

    x = x_r[...]
    h = h_r[...]
    c = c_r[...]
    gates = (lax.dot_general(x, wih_r[...], (((1,), (1,)), ((), ())),
                             preferred_element_type=jnp.float32)
             + lax.dot_general(h, whh_r[...], (((1,), (1,)), ((), ())),
                               preferred_element_type=jnp.float32)
             + b1_r[...] + b2_r[...])
    ig = jax.nn.sigmoid(gates[:, 0:H])
    fg = jax.nn.sigmoid(gates[:, H:2 * H])
    gg = jnp.tanh(gates[:, 2 * H:3 * H])
    og = jax.nn.sigmoid(gates[:, 3 * H:4 * H])
    c1 = fg * c + ig * gg
    h1 = og * jnp.tanh(c1)

    m = mrow_r[...] > 0
    xo_r[...] = jnp.where(special, x, jnp.where(m, h1, 0.0))
    hm_r[...] = jnp.where(m, h1, h)
    cm_r[...] = jnp.where(m, c1, c)


def _tc_lstm(x2, h2, c2, mrow, m2d, wih, whh, b1, b2):
    full = lambda shape: pl.BlockSpec(shape, lambda i: (0, 0))
    blk = pl.BlockSpec((BLK, 128), lambda i: (i, 0))
    return pl.pallas_call(
        _tc_body,
        grid=(NB,),
        in_specs=[
            blk, blk, blk,
            pl.BlockSpec((BLK, 1), lambda i: (i, 0)),
            full((MROWS, MCOLS)),
            full((G, D)), full((G, H)),
            full((1, G)), full((1, G)),
        ],
        out_specs=[blk, blk, blk, full((MROWS, MCOLS))],
        out_shape=[
            jax.ShapeDtypeStruct((B, H), jnp.float32),
            jax.ShapeDtypeStruct((B, H), jnp.float32),
            jax.ShapeDtypeStruct((B, H), jnp.float32),
            jax.ShapeDtypeStruct((MROWS, MCOLS), jnp.int32),
        ],
        scratch_shapes=[pltpu.SMEM((1,), jnp.float32)],
    )(x2, h2, c2, mrow, m2d, wih, whh, b1, b2)


def _sc_scatter(hm, cm, pos2d):
    mesh = plsc.VectorSubcoreMesh(core_axis_name="c", subcore_axis_name="s")
    nw = mesh.num_cores * mesh.num_subcores
    rpw = B // nw
    chunks = rpw // 128

    @functools.partial(
        pl.kernel,
        out_type=(
            jax.ShapeDtypeStruct((B, H), jnp.float32),
            jax.ShapeDtypeStruct((B, H), jnp.float32),
        ),
        mesh=mesh,
        scratch_types=[
            pltpu.VMEM((chunks, 128), jnp.int32),
            pltpu.VMEM((128, H), jnp.float32),
            pltpu.VMEM((128, H), jnp.float32),
            pltpu.SemaphoreType.DMA,
        ],
    )
    def scatter(hm_hbm, cm_hbm, pos_hbm, hout_hbm, cout_hbm,
                pos_v, hrows, crows, sem):
        wid = lax.axis_index("s") * mesh.num_cores + lax.axis_index("c")
        base = wid * rpw
        pltpu.sync_copy(pos_hbm.at[pl.ds(wid * chunks, chunks)], pos_v)
        for j in range(chunks):
            pltpu.sync_copy(hm_hbm.at[pl.ds(base + j * 128, 128)], hrows)
            pltpu.sync_copy(cm_hbm.at[pl.ds(base + j * 128, 128)], crows)
            cp_h = pltpu.async_copy(hrows, hout_hbm.at[pos_v.at[j]], sem)
            cp_c = pltpu.async_copy(crows, cout_hbm.at[pos_v.at[j]], sem)
            cp_h.wait()
            cp_c.wait()

    return scatter(hm, cm, pos2d)


def kernel(x, mask, h0, c0, W_ih, W_hh, b_ih, b_hh):
    x2 = x.reshape(B, D)
    h2 = h0.reshape(B, H)
    c2 = c0.reshape(B, H)
    mrow = mask.reshape(B, 1)
    m2d = mask.reshape(MROWS, MCOLS)
    b1 = b_ih.reshape(1, G)
    b2 = b_hh.reshape(1, G)

    xo, hm, cm, pos2d = _tc_lstm(x2, h2, c2, mrow, m2d, W_ih, W_hh, b1, b2)
    h_out, c_out = _sc_scatter(hm, cm, pos2d)

    return (xo.reshape(B, 1, H),
            h_out.reshape(1, B, H),
            c_out.reshape(1, B, H))


# speedup vs baseline: 2.6907x; 1.0144x over previous
import functools

import jax
import jax.numpy as jnp
from jax import lax
from jax.experimental import pallas as pl
from jax.experimental.pallas import tpu as pltpu
from jax.experimental.pallas import tpu_sc as plsc

B = 16384
D = 128
H = 128
G = 4 * H
BLK = 1024
NB = B // BLK
MROWS = 128
MCOLS = B // MROWS


def _sigmoid(z):
    return 0.5 * jnp.tanh(z * 0.5) + 0.5


def _tc_body(x_r, h_r, c_r, mrow_r, m2d_r, w_r, b1_r, b2_r,
             xo_r, hm_r, cm_r, pos_r, valid_s):
    i = pl.program_id(0)

    @pl.when(i == 0)
    def _compute_pos():
        mf = m2d_r[...].astype(jnp.float32)
        row = lax.broadcasted_iota(jnp.int32, (MROWS, MCOLS), 0).astype(jnp.float32)
        col = lax.broadcasted_iota(jnp.int32, (MROWS, MCOLS), 1).astype(jnp.float32)
        lt = (row <= col).astype(jnp.float32)
        sl = (col < row).astype(jnp.float32)
        t = lax.dot_general(mf, lt, (((1,), (0,)), ((), ())),
                            preferred_element_type=jnp.float32)
        rowsum = t[:, MCOLS - 1:MCOLS]
        rowoff = lax.dot_general(sl, rowsum, (((1,), (0,)), ((), ())),
                                 preferred_element_type=jnp.float32)
        a = t + rowoff
        valid = jnp.sum(mf)
        valid_s[0] = valid
        bidx = row * float(MCOLS) + col
        posf = jnp.where(mf > 0.0, a - 1.0, valid + bidx - a)
        pos_r[...] = posf.astype(jnp.int32)

    valid = valid_s[0]
    special = jnp.logical_or(valid == 0.0, valid == float(B))

    x = x_r[...]
    h = h_r[...]
    c = c_r[...]
    xh = jnp.concatenate([x, h], axis=1)
    gates = (lax.dot_general(xh, w_r[...], (((1,), (1,)), ((), ())),
                             preferred_element_type=jnp.float32)
             + b1_r[...] + b2_r[...])
    ig = _sigmoid(gates[:, 0:H])
    fg = _sigmoid(gates[:, H:2 * H])
    gg = jnp.tanh(gates[:, 2 * H:3 * H])
    og = _sigmoid(gates[:, 3 * H:4 * H])
    c1 = fg * c + ig * gg
    h1 = og * jnp.tanh(c1)

    m = mrow_r[...] > 0
    xo_r[...] = jnp.where(special, x, jnp.where(m, h1, 0.0))
    hm_r[...] = jnp.where(m, h1, h)
    cm_r[...] = jnp.where(m, c1, c)


def _tc_lstm(x2, h2, c2, mrow, m2d, w, b1, b2):
    full = lambda shape: pl.BlockSpec(shape, lambda i: (0, 0))
    blk = pl.BlockSpec((BLK, 128), lambda i: (i, 0))
    return pl.pallas_call(
        _tc_body,
        grid=(NB,),
        in_specs=[
            blk, blk, blk,
            pl.BlockSpec((BLK, 1), lambda i: (i, 0)),
            full((MROWS, MCOLS)),
            full((G, D + H)),
            full((1, G)), full((1, G)),
        ],
        out_specs=[blk, blk, blk, full((MROWS, MCOLS))],
        out_shape=[
            jax.ShapeDtypeStruct((B, H), jnp.float32),
            jax.ShapeDtypeStruct((B, H), jnp.float32),
            jax.ShapeDtypeStruct((B, H), jnp.float32),
            jax.ShapeDtypeStruct((MROWS, MCOLS), jnp.int32),
        ],
        scratch_shapes=[pltpu.SMEM((1,), jnp.float32)],
    )(x2, h2, c2, mrow, m2d, w, b1, b2)


def _sc_scatter(hm, cm, pos2d):
    mesh = plsc.VectorSubcoreMesh(core_axis_name="c", subcore_axis_name="s")
    nw = mesh.num_cores * mesh.num_subcores
    rpw = B // nw
    chunks = rpw // 128

    @functools.partial(
        pl.kernel,
        out_type=(
            jax.ShapeDtypeStruct((B, H), jnp.float32),
            jax.ShapeDtypeStruct((B, H), jnp.float32),
        ),
        mesh=mesh,
        scratch_types=[
            pltpu.VMEM((chunks, 128), jnp.int32),
            pltpu.VMEM((128, H), jnp.float32),
            pltpu.VMEM((128, H), jnp.float32),
            pltpu.SemaphoreType.DMA,
        ],
    )
    def scatter(hm_hbm, cm_hbm, pos_hbm, hout_hbm, cout_hbm,
                pos_v, hrows, crows, sem):
        wid = lax.axis_index("s") * mesh.num_cores + lax.axis_index("c")
        base = wid * rpw
        pltpu.sync_copy(pos_hbm.at[pl.ds(wid * chunks, chunks)], pos_v)
        for j in range(chunks):
            pltpu.sync_copy(hm_hbm.at[pl.ds(base + j * 128, 128)], hrows)
            pltpu.sync_copy(cm_hbm.at[pl.ds(base + j * 128, 128)], crows)
            cp_h = pltpu.async_copy(hrows, hout_hbm.at[pos_v.at[j]], sem)
            cp_c = pltpu.async_copy(crows, cout_hbm.at[pos_v.at[j]], sem)
            cp_h.wait()
            cp_c.wait()

    return scatter(hm, cm, pos2d)


def kernel(x, mask, h0, c0, W_ih, W_hh, b_ih, b_hh):
    x2 = x.reshape(B, D)
    h2 = h0.reshape(B, H)
    c2 = c0.reshape(B, H)
    mrow = mask.reshape(B, 1)
    m2d = mask.reshape(MROWS, MCOLS)
    b1 = b_ih.reshape(1, G)
    b2 = b_hh.reshape(1, G)
    w = jnp.concatenate([W_ih, W_hh], axis=1)

    xo, hm, cm, pos2d = _tc_lstm(x2, h2, c2, mrow, m2d, w, b1, b2)
    h_out, c_out = _sc_scatter(hm, cm, pos2d)

    return (xo.reshape(B, 1, H),
            h_out.reshape(1, B, H),
            c_out.reshape(1, B, H))


# speedup vs baseline: 2.9389x; 1.0922x over previous
import functools

import jax
import jax.numpy as jnp
from jax import lax
from jax.experimental import pallas as pl
from jax.experimental.pallas import tpu as pltpu
from jax.experimental.pallas import tpu_sc as plsc

B = 16384
D = 128
H = 128
G = 4 * H
BLK = 1024
NB = B // BLK
MROWS = 128
MCOLS = B // MROWS


def _sigmoid(z):
    return 0.5 * jnp.tanh(z * 0.5) + 0.5


def _tc_body(x_r, h_r, c_r, mblk_r, m2d_r, w_r, b1_r, b2_r,
             xo_r, hm_r, cm_r, pos_r, valid_s):
    i = pl.program_id(0)

    @pl.when(i == 0)
    def _compute_pos():
        mf = m2d_r[...].astype(jnp.float32)
        row = lax.broadcasted_iota(jnp.int32, (MROWS, MCOLS), 0).astype(jnp.float32)
        col = lax.broadcasted_iota(jnp.int32, (MROWS, MCOLS), 1).astype(jnp.float32)
        lt = (row <= col).astype(jnp.float32)
        sl = (col < row).astype(jnp.float32)
        t = lax.dot_general(mf, lt, (((1,), (0,)), ((), ())),
                            preferred_element_type=jnp.float32)
        rowsum = t[:, MCOLS - 1:MCOLS]
        rowoff = lax.dot_general(sl, rowsum, (((1,), (0,)), ((), ())),
                                 preferred_element_type=jnp.float32)
        a = t + rowoff
        valid = jnp.sum(mf)
        valid_s[0] = valid
        bidx = row * float(MCOLS) + col
        posf = jnp.where(mf > 0.0, a - 1.0, valid + bidx - a)
        pos_r[...] = posf.astype(jnp.int32)

    valid = valid_s[0]
    special = jnp.logical_or(valid == 0.0, valid == float(B))

    x = x_r[...]
    h = h_r[...]
    c = c_r[...]
    xh = jnp.concatenate([x, h], axis=1)
    gates = (lax.dot_general(xh, w_r[...], (((1,), (1,)), ((), ())),
                             preferred_element_type=jnp.float32)
             + b1_r[...] + b2_r[...])
    ig = _sigmoid(gates[:, 0:H])
    fg = _sigmoid(gates[:, H:2 * H])
    gg = jnp.tanh(gates[:, 2 * H:3 * H])
    og = _sigmoid(gates[:, 3 * H:4 * H])
    c1 = fg * c + ig * gg
    h1 = og * jnp.tanh(c1)

    mt = jnp.transpose(mblk_r[...], (1, 0))
    nsub = BLK // 128
    for j in range(nsub):
        sl = slice(j * 128, (j + 1) * 128)
        m = mt[:, j:j + 1] > 0
        xo_r[sl, :] = jnp.where(special, x[sl, :],
                                jnp.where(m, h1[sl, :], 0.0))
        hm_r[sl, :] = jnp.where(m, h1[sl, :], h[sl, :])
        cm_r[sl, :] = jnp.where(m, c1[sl, :], c[sl, :])


def _tc_lstm(x2, h2, c2, m2d, w, b1, b2):
    full = lambda shape: pl.BlockSpec(shape, lambda i: (0, 0))
    blk = pl.BlockSpec((BLK, 128), lambda i: (i, 0))
    return pl.pallas_call(
        _tc_body,
        grid=(NB,),
        in_specs=[
            blk, blk, blk,
            pl.BlockSpec((BLK // 128, MCOLS), lambda i: (i, 0)),
            full((MROWS, MCOLS)),
            full((G, D + H)),
            full((1, G)), full((1, G)),
        ],
        out_specs=[blk, blk, blk, full((MROWS, MCOLS))],
        out_shape=[
            jax.ShapeDtypeStruct((B, H), jnp.float32),
            jax.ShapeDtypeStruct((B, H), jnp.float32),
            jax.ShapeDtypeStruct((B, H), jnp.float32),
            jax.ShapeDtypeStruct((MROWS, MCOLS), jnp.int32),
        ],
        scratch_shapes=[pltpu.SMEM((1,), jnp.float32)],
    )(x2, h2, c2, m2d, m2d, w, b1, b2)


def _sc_scatter(hm, cm, pos2d):
    mesh = plsc.VectorSubcoreMesh(core_axis_name="c", subcore_axis_name="s")
    nw = mesh.num_cores * mesh.num_subcores
    rpw = B // nw
    chunks = rpw // 128

    @functools.partial(
        pl.kernel,
        out_type=(
            jax.ShapeDtypeStruct((B, H), jnp.float32),
            jax.ShapeDtypeStruct((B, H), jnp.float32),
        ),
        mesh=mesh,
        scratch_types=[
            pltpu.VMEM((chunks, 128), jnp.int32),
            pltpu.VMEM((128, H), jnp.float32),
            pltpu.VMEM((128, H), jnp.float32),
            pltpu.SemaphoreType.DMA,
        ],
    )
    def scatter(hm_hbm, cm_hbm, pos_hbm, hout_hbm, cout_hbm,
                pos_v, hrows, crows, sem):
        wid = lax.axis_index("s") * mesh.num_cores + lax.axis_index("c")
        base = wid * rpw
        pltpu.sync_copy(pos_hbm.at[pl.ds(wid * chunks, chunks)], pos_v)
        for j in range(chunks):
            pltpu.sync_copy(hm_hbm.at[pl.ds(base + j * 128, 128)], hrows)
            pltpu.sync_copy(cm_hbm.at[pl.ds(base + j * 128, 128)], crows)
            cp_h = pltpu.async_copy(hrows, hout_hbm.at[pos_v.at[j]], sem)
            cp_c = pltpu.async_copy(crows, cout_hbm.at[pos_v.at[j]], sem)
            cp_h.wait()
            cp_c.wait()

    return scatter(hm, cm, pos2d)


def kernel(x, mask, h0, c0, W_ih, W_hh, b_ih, b_hh):
    x2 = x.reshape(B, D)
    h2 = h0.reshape(B, H)
    c2 = c0.reshape(B, H)
    m2d = mask.reshape(MROWS, MCOLS)
    b1 = b_ih.reshape(1, G)
    b2 = b_hh.reshape(1, G)
    w = jnp.concatenate([W_ih, W_hh], axis=1)

    xo, hm, cm, pos2d = _tc_lstm(x2, h2, c2, m2d, w, b1, b2)
    h_out, c_out = _sc_scatter(hm, cm, pos2d)

    return (xo.reshape(B, 1, H),
            h_out.reshape(1, B, H),
            c_out.reshape(1, B, H))


# speedup vs baseline: 3.0543x; 1.0393x over previous
import functools

import jax
import jax.numpy as jnp
from jax import lax
from jax.experimental import pallas as pl
from jax.experimental.pallas import tpu as pltpu
from jax.experimental.pallas import tpu_sc as plsc

B = 16384
D = 128
H = 128
G = 4 * H
BLK = 1024
HB = B // 2
NBH = HB // BLK
MROWS = 128
MCOLS = B // MROWS


def _sigmoid(z):
    return 0.5 * jnp.tanh(z * 0.5) + 0.5


def _tc_body_a(x_r, h_r, c_r, mblk_r, m2d_r, w_r, b1_r, b2_r, xo_al_r,
               xo_r, hm_r, cm_r, pos_r, validv_r, valid_s):
    i = pl.program_id(0)

    @pl.when(i == 0)
    def _compute_pos():
        mf = m2d_r[...].astype(jnp.float32)
        row = lax.broadcasted_iota(jnp.int32, (MROWS, MCOLS), 0).astype(jnp.float32)
        col = lax.broadcasted_iota(jnp.int32, (MROWS, MCOLS), 1).astype(jnp.float32)
        lt = (row <= col).astype(jnp.float32)
        sl = (col < row).astype(jnp.float32)
        t = lax.dot_general(mf, lt, (((1,), (0,)), ((), ())),
                            preferred_element_type=jnp.float32)
        rowsum = t[:, MCOLS - 1:MCOLS]
        rowoff = lax.dot_general(sl, rowsum, (((1,), (0,)), ((), ())),
                                 preferred_element_type=jnp.float32)
        a = t + rowoff
        valid = jnp.sum(mf)
        valid_s[0] = valid
        validv_r[...] = jnp.reshape(valid, (1, 1))
        bidx = row * float(MCOLS) + col
        posf = jnp.where(mf > 0.0, a - 1.0, valid + bidx - a)
        pos_r[...] = posf.astype(jnp.int32)

    _lstm_block(x_r, h_r, c_r, mblk_r, w_r, b1_r, b2_r,
                xo_r, hm_r, cm_r, valid_s[0])


def _tc_body_b(x_r, h_r, c_r, mblk_r, w_r, b1_r, b2_r, validv_r, xo_al_r,
               xo_r, hm_r, cm_r):
    _lstm_block(x_r, h_r, c_r, mblk_r, w_r, b1_r, b2_r,
                xo_r, hm_r, cm_r, validv_r[...])


def _lstm_block(x_r, h_r, c_r, mblk_r, w_r, b1_r, b2_r,
                xo_r, hm_r, cm_r, valid):
    special = jnp.logical_or(valid == 0.0, valid == float(B))
    x = x_r[...]
    h = h_r[...]
    c = c_r[...]
    xh = jnp.concatenate([x, h], axis=1)
    gates = (lax.dot_general(xh, w_r[...], (((1,), (1,)), ((), ())),
                             preferred_element_type=jnp.float32)
             + b1_r[...] + b2_r[...])
    ig = _sigmoid(gates[:, 0:H])
    fg = _sigmoid(gates[:, H:2 * H])
    gg = jnp.tanh(gates[:, 2 * H:3 * H])
    og = _sigmoid(gates[:, 3 * H:4 * H])
    c1 = fg * c + ig * gg
    h1 = og * jnp.tanh(c1)

    mt = jnp.transpose(mblk_r[...], (1, 0))
    nsub = BLK // 128
    for j in range(nsub):
        sl = slice(j * 128, (j + 1) * 128)
        m = mt[:, j:j + 1] > 0
        xo_r[sl, :] = jnp.where(special, x[sl, :],
                                jnp.where(m, h1[sl, :], 0.0))
        hm_r[sl, :] = jnp.where(m, h1[sl, :], h[sl, :])
        cm_r[sl, :] = jnp.where(m, c1[sl, :], c[sl, :])


def _tc_half(half, x2, h2, c2, m2d, w, b1, b2, xo_in, validv=None):
    off = half * NBH
    full = lambda shape: pl.BlockSpec(shape, lambda i: (0, 0))
    gblk = pl.BlockSpec((BLK, 128), lambda i: (i + off, 0))
    hblk = pl.BlockSpec((BLK, 128), lambda i: (i, 0))
    mblk = pl.BlockSpec((BLK // 128, MCOLS), lambda i: (i + off, 0))
    half_out = [
        jax.ShapeDtypeStruct((B, H), jnp.float32),
        jax.ShapeDtypeStruct((HB, H), jnp.float32),
        jax.ShapeDtypeStruct((HB, H), jnp.float32),
    ]
    anyspec = pl.BlockSpec(memory_space=pl.ANY)
    if half == 0:
        return pl.pallas_call(
            _tc_body_a,
            grid=(NBH,),
            in_specs=[gblk, gblk, gblk, mblk, full((MROWS, MCOLS)),
                      full((G, D + H)), full((1, G)), full((1, G)), anyspec],
            out_specs=[gblk, hblk, hblk, full((MROWS, MCOLS)),
                       full((1, 1))],
            out_shape=half_out + [
                jax.ShapeDtypeStruct((MROWS, MCOLS), jnp.int32),
                jax.ShapeDtypeStruct((1, 1), jnp.float32),
            ],
            input_output_aliases={8: 0},
            scratch_shapes=[pltpu.SMEM((1,), jnp.float32)],
        )(x2, h2, c2, m2d, m2d, w, b1, b2, xo_in)
    return pl.pallas_call(
        _tc_body_b,
        grid=(NBH,),
        in_specs=[gblk, gblk, gblk, mblk,
                  full((G, D + H)), full((1, G)), full((1, G)),
                  full((1, 1)), anyspec],
        out_specs=[gblk, hblk, hblk],
        out_shape=half_out,
        input_output_aliases={8: 0},
        scratch_shapes=[],
    )(x2, h2, c2, m2d, w, b1, b2, validv, xo_in)


def _sc_scatter_half(half, hm, cm, pos2d, h_ref, c_ref):
    mesh = plsc.VectorSubcoreMesh(core_axis_name="c", subcore_axis_name="s")
    nw = mesh.num_cores * mesh.num_subcores
    rpw = HB // nw
    chunks = rpw // 128
    prow_off = half * (HB // MCOLS)

    @functools.partial(
        pl.kernel,
        out_type=(),
        mesh=mesh,
        scratch_types=[
            pltpu.VMEM((chunks, 128), jnp.int32),
            pltpu.VMEM((128, H), jnp.float32),
            pltpu.VMEM((128, H), jnp.float32),
            pltpu.SemaphoreType.DMA,
        ],
    )
    def scatter(hm_hbm, cm_hbm, pos_hbm, hout_hbm, cout_hbm,
                pos_v, hrows, crows, sem):
        wid = lax.axis_index("s") * mesh.num_cores + lax.axis_index("c")
        base = wid * rpw
        pltpu.sync_copy(pos_hbm.at[pl.ds(prow_off + wid * chunks, chunks)],
                        pos_v)
        for j in range(chunks):
            pltpu.sync_copy(hm_hbm.at[pl.ds(base + j * 128, 128)], hrows)
            pltpu.sync_copy(cm_hbm.at[pl.ds(base + j * 128, 128)], crows)
            cp_h = pltpu.async_copy(hrows, hout_hbm.at[pos_v.at[j]], sem)
            cp_c = pltpu.async_copy(crows, cout_hbm.at[pos_v.at[j]], sem)
            cp_h.wait()
            cp_c.wait()

    scatter(hm, cm, pos2d, h_ref, c_ref)


def kernel(x, mask, h0, c0, W_ih, W_hh, b_ih, b_hh):
    x2 = x.reshape(B, D)
    h2 = h0.reshape(B, H)
    c2 = c0.reshape(B, H)
    m2d = mask.reshape(MROWS, MCOLS)
    b1 = b_ih.reshape(1, G)
    b2 = b_hh.reshape(1, G)
    w = jnp.concatenate([W_ih, W_hh], axis=1)

    xo0 = pl.empty((B, H), jnp.float32)
    xo1, hm_a, cm_a, pos2d, validv = _tc_half(0, x2, h2, c2, m2d, w, b1, b2,
                                              xo0)
    xo, hm_b, cm_b = _tc_half(1, x2, h2, c2, m2d, w, b1, b2, xo1, validv)

    h_ref = jax.new_ref(pl.empty((B, H), jnp.float32))
    c_ref = jax.new_ref(pl.empty((B, H), jnp.float32))
    _sc_scatter_half(0, hm_a, cm_a, pos2d, h_ref, c_ref)
    _sc_scatter_half(1, hm_b, cm_b, pos2d, h_ref, c_ref)
    h_out = jax.freeze(h_ref)
    c_out = jax.freeze(c_ref)

    return (xo.reshape(B, 1, H),
            h_out.reshape(1, B, H),
            c_out.reshape(1, B, H))
